# Initial kernel scaffold; baseline (speedup 1.0000x reference)
import jax
import jax.numpy as jnp
from jax.experimental import pallas as pl


def kernel(x, cmap):
    raise NotImplementedError("write your pallas kernel here")


# speedup vs baseline: 70.2497x; 70.2497x over previous
import functools

import jax
import jax.numpy as jnp
from jax import lax
from jax.experimental import pallas as pl
from jax.experimental.pallas import tpu as pltpu
from jax.experimental.pallas import tpu_sc as plsc

B, H, W = 16, 512, 512
P = H * W
NW = 32
PIX_PER_W = B * P // NW
CHUNK = 16384
N_CHUNKS = PIX_PER_W // CHUNK
L = 16
NCOLORS = 2048


def _sc_colorize(x2, cmapt):
    mesh = plsc.VectorSubcoreMesh(core_axis_name="c", subcore_axis_name="s")

    @functools.partial(
        pl.kernel,
        out_type=jax.ShapeDtypeStruct((B * 3 * P,), jnp.float32),
        mesh=mesh,
        compiler_params=pltpu.CompilerParams(needs_layout_passes=False),
        scratch_types=[
            pltpu.VMEM((NCOLORS,), jnp.float32),
            pltpu.VMEM((NCOLORS,), jnp.float32),
            pltpu.VMEM((NCOLORS,), jnp.float32),
            pltpu.VMEM((CHUNK,), jnp.int32),
            pltpu.VMEM((CHUNK,), jnp.float32),
            pltpu.VMEM((CHUNK,), jnp.float32),
            pltpu.VMEM((CHUNK,), jnp.float32),
        ],
    )
    def k(x_hbm, cmapt_hbm, out_hbm, rtab, gtab, btab, idxv, rbuf, gbuf, bbuf):
        wid = lax.axis_index("s") * 2 + lax.axis_index("c")
        b = wid // 2
        base = (wid % 2) * PIX_PER_W

        pltpu.sync_copy(cmapt_hbm.at[pl.ds(0, NCOLORS)], rtab)
        pltpu.sync_copy(cmapt_hbm.at[pl.ds(NCOLORS, NCOLORS)], gtab)
        pltpu.sync_copy(cmapt_hbm.at[pl.ds(2 * NCOLORS, NCOLORS)], btab)

        for c in range(N_CHUNKS):
            off = base + c * CHUNK
            pltpu.sync_copy(x_hbm.at[pl.ds(b * P + off, CHUNK)], idxv)

            def body(i, carry):
                idx = idxv[pl.ds(i * L, L)]
                rbuf[pl.ds(i * L, L)] = plsc.load_gather(rtab, [idx])
                gbuf[pl.ds(i * L, L)] = plsc.load_gather(gtab, [idx])
                bbuf[pl.ds(i * L, L)] = plsc.load_gather(btab, [idx])
                return carry

            lax.fori_loop(0, CHUNK // L, body, 0)

            obase = b * 3 * P + off
            pltpu.sync_copy(rbuf, out_hbm.at[pl.ds(obase, CHUNK)])
            pltpu.sync_copy(gbuf, out_hbm.at[pl.ds(obase + P, CHUNK)])
            pltpu.sync_copy(bbuf, out_hbm.at[pl.ds(obase + 2 * P, CHUNK)])

    return k(x2, cmapt)


def kernel(x, cmap):
    x2 = x.reshape(-1)
    cmapt = cmap.T.reshape(-1)
    out = _sc_colorize(x2, cmapt)
    return out.reshape(B, 3, H, W)


# speedup vs baseline: 112.2242x; 1.5975x over previous
import functools

import jax
import jax.numpy as jnp
from jax import lax
from jax.experimental import pallas as pl
from jax.experimental.pallas import tpu as pltpu
from jax.experimental.pallas import tpu_sc as plsc

B, H, W = 16, 512, 512
P = H * W
NW = 32
PIX_PER_W = B * P // NW
CHUNK = 16384
N_CHUNKS = PIX_PER_W // CHUNK
L = 16
NCOLORS = 2048


def _sc_colorize(x2, cmapt):
    mesh = plsc.VectorSubcoreMesh(core_axis_name="c", subcore_axis_name="s")

    @functools.partial(
        pl.kernel,
        out_type=jax.ShapeDtypeStruct((B * 3 * P,), jnp.float32),
        mesh=mesh,
        compiler_params=pltpu.CompilerParams(needs_layout_passes=False),
        scratch_types=[
            pltpu.VMEM((NCOLORS,), jnp.float32),
            pltpu.VMEM((NCOLORS,), jnp.float32),
            pltpu.VMEM((NCOLORS,), jnp.float32),
            pltpu.VMEM((CHUNK,), jnp.int32),
            pltpu.VMEM((CHUNK,), jnp.float32),
            pltpu.VMEM((CHUNK,), jnp.float32),
            pltpu.VMEM((CHUNK,), jnp.float32),
        ],
    )
    def k(x_hbm, cmapt_hbm, out_hbm, rtab, gtab, btab, idxv, rbuf, gbuf, bbuf):
        wid = lax.axis_index("s") * 2 + lax.axis_index("c")
        b = wid // 2
        base = (wid % 2) * PIX_PER_W

        pltpu.sync_copy(cmapt_hbm.at[pl.ds(0, NCOLORS)], rtab)
        pltpu.sync_copy(cmapt_hbm.at[pl.ds(NCOLORS, NCOLORS)], gtab)
        pltpu.sync_copy(cmapt_hbm.at[pl.ds(2 * NCOLORS, NCOLORS)], btab)

        for c in range(N_CHUNKS):
            off = base + c * CHUNK
            pltpu.sync_copy(x_hbm.at[pl.ds(b * P + off, CHUNK)], idxv)

            @plsc.parallel_loop(0, CHUNK, step=L, unroll=8)
            def body(i):
                idx = idxv[pl.ds(i, L)]
                rbuf[pl.ds(i, L)] = plsc.load_gather(rtab, [idx])
                gbuf[pl.ds(i, L)] = plsc.load_gather(gtab, [idx])
                bbuf[pl.ds(i, L)] = plsc.load_gather(btab, [idx])

            obase = b * 3 * P + off
            pltpu.sync_copy(rbuf, out_hbm.at[pl.ds(obase, CHUNK)])
            pltpu.sync_copy(gbuf, out_hbm.at[pl.ds(obase + P, CHUNK)])
            pltpu.sync_copy(bbuf, out_hbm.at[pl.ds(obase + 2 * P, CHUNK)])

    return k(x2, cmapt)


def kernel(x, cmap):
    x2 = x.reshape(-1)
    cmapt = cmap.T.reshape(-1)
    out = _sc_colorize(x2, cmapt)
    return out.reshape(B, 3, H, W)


# speedup vs baseline: 131.3362x; 1.1703x over previous
import functools

import jax
import jax.numpy as jnp
from jax import lax
from jax.experimental import pallas as pl
from jax.experimental.pallas import tpu as pltpu
from jax.experimental.pallas import tpu_sc as plsc

B, H, W = 16, 512, 512
P = H * W
NW = 32
PIX_PER_W = B * P // NW
CHUNK = 8192
N_CHUNKS = PIX_PER_W // CHUNK
L = 16
NCOLORS = 2048


def _sc_colorize(x2, cmapt):
    mesh = plsc.VectorSubcoreMesh(core_axis_name="c", subcore_axis_name="s")

    @functools.partial(
        pl.kernel,
        out_type=jax.ShapeDtypeStruct((B * 3 * P,), jnp.float32),
        mesh=mesh,
        compiler_params=pltpu.CompilerParams(needs_layout_passes=False),
        scratch_types=[
            pltpu.VMEM((NCOLORS,), jnp.float32),
            pltpu.VMEM((NCOLORS,), jnp.float32),
            pltpu.VMEM((NCOLORS,), jnp.float32),
            [pltpu.VMEM((CHUNK,), jnp.int32)] * 2,
            [pltpu.VMEM((CHUNK,), jnp.float32)] * 2,
            [pltpu.VMEM((CHUNK,), jnp.float32)] * 2,
            [pltpu.VMEM((CHUNK,), jnp.float32)] * 2,
            [pltpu.SemaphoreType.DMA] * 2,
            [pltpu.SemaphoreType.DMA] * 2,
        ],
    )
    def k(x_hbm, cmapt_hbm, out_hbm, rtab, gtab, btab, idxv, rbuf, gbuf,
          bbuf, insem, outsem):
        wid = lax.axis_index("s") * 2 + lax.axis_index("c")
        b = wid // 2
        base = (wid % 2) * PIX_PER_W

        pltpu.sync_copy(cmapt_hbm.at[pl.ds(0, NCOLORS)], rtab)
        pltpu.sync_copy(cmapt_hbm.at[pl.ds(NCOLORS, NCOLORS)], gtab)
        pltpu.sync_copy(cmapt_hbm.at[pl.ds(2 * NCOLORS, NCOLORS)], btab)

        in_desc = [None, None]
        out_descs = [None, None]

        def start_in(c):
            s = c & 1
            in_desc[s] = pltpu.async_copy(
                x_hbm.at[pl.ds(b * P + base + c * CHUNK, CHUNK)],
                idxv[s], insem[s])

        start_in(0)
        for c in range(N_CHUNKS):
            s = c & 1
            if c + 1 < N_CHUNKS:
                start_in(c + 1)
            in_desc[s].wait()
            if out_descs[s] is not None:
                for d in out_descs[s]:
                    d.wait()

            @plsc.parallel_loop(0, CHUNK, step=L, unroll=8)
            def body(i):
                idx = idxv[s][pl.ds(i, L)]
                rbuf[s][pl.ds(i, L)] = plsc.load_gather(rtab, [idx])
                gbuf[s][pl.ds(i, L)] = plsc.load_gather(gtab, [idx])
                bbuf[s][pl.ds(i, L)] = plsc.load_gather(btab, [idx])

            obase = b * 3 * P + base + c * CHUNK
            out_descs[s] = [
                pltpu.async_copy(rbuf[s], out_hbm.at[pl.ds(obase, CHUNK)],
                                 outsem[s]),
                pltpu.async_copy(gbuf[s], out_hbm.at[pl.ds(obase + P, CHUNK)],
                                 outsem[s]),
                pltpu.async_copy(bbuf[s],
                                 out_hbm.at[pl.ds(obase + 2 * P, CHUNK)],
                                 outsem[s]),
            ]
        for ds_ in out_descs:
            if ds_ is not None:
                for d in ds_:
                    d.wait()

    return k(x2, cmapt)


def kernel(x, cmap):
    x2 = x.reshape(-1)
    cmapt = cmap.T.reshape(-1)
    out = _sc_colorize(x2, cmapt)
    return out.reshape(B, 3, H, W)


# speedup vs baseline: 135.1186x; 1.0288x over previous
import functools

import jax
import jax.numpy as jnp
from jax import lax
from jax.experimental import pallas as pl
from jax.experimental.pallas import tpu as pltpu
from jax.experimental.pallas import tpu_sc as plsc

B, H, W = 16, 512, 512
NW = 32
TPI = (H // 8) * (W // 128)
TS = 8
SLAB = TS * 8 * 128
TILES_PER_WORKER = B * TPI // NW
N_SLABS = TILES_PER_WORKER // TS
L = 16
NCOLORS = 2048


def _sc_colorize(x4, cmapf):
    mesh = plsc.VectorSubcoreMesh(core_axis_name="c", subcore_axis_name="s")

    @functools.partial(
        pl.kernel,
        out_type=jax.ShapeDtypeStruct((B * 3 * TPI, 8, 128), jnp.float32),
        mesh=mesh,
        compiler_params=pltpu.CompilerParams(needs_layout_passes=False),
        scratch_types=[
            pltpu.VMEM((3 * NCOLORS,), jnp.float32),
            [pltpu.VMEM((TS, 8, 128), jnp.int32)] * 2,
            [pltpu.VMEM((TS, 8, 128), jnp.float32)] * 2,
            [pltpu.VMEM((TS, 8, 128), jnp.float32)] * 2,
            [pltpu.VMEM((TS, 8, 128), jnp.float32)] * 2,
            [pltpu.SemaphoreType.DMA] * 2,
            [pltpu.SemaphoreType.DMA] * 2,
        ],
    )
    def k(x_hbm, cmap_hbm, out_hbm, tab, idxv, rbuf, gbuf, bbuf, insem,
          outsem):
        wid = lax.axis_index("s") * 2 + lax.axis_index("c")
        b = wid // 2
        gbase = b * TPI + (wid % 2) * TILES_PER_WORKER

        pltpu.sync_copy(cmap_hbm, tab)

        in_desc = [None, None]
        out_descs = [None, None]

        def start_in(c):
            s = c & 1
            in_desc[s] = pltpu.async_copy(
                x_hbm.at[pl.ds(gbase + c * TS, TS), :, :], idxv[s], insem[s])

        start_in(0)
        for c in range(N_SLABS):
            s = c & 1
            if c + 1 < N_SLABS:
                start_in(c + 1)
            in_desc[s].wait()
            if out_descs[s] is not None:
                for d in out_descs[s]:
                    d.wait()

            @plsc.parallel_loop(0, SLAB, step=L, unroll=8)
            def body(i):
                t = i // 1024
                r = (i // 128) % 8
                cc = i % 128
                i3 = idxv[s][t, r, pl.ds(cc, L)] * 3
                rbuf[s][t, r, pl.ds(cc, L)] = plsc.load_gather(tab, [i3])
                gbuf[s][t, r, pl.ds(cc, L)] = plsc.load_gather(tab, [i3 + 1])
                bbuf[s][t, r, pl.ds(cc, L)] = plsc.load_gather(tab, [i3 + 2])

            off = (wid % 2) * TILES_PER_WORKER + c * TS
            out_descs[s] = [
                pltpu.async_copy(
                    rbuf[s],
                    out_hbm.at[pl.ds((b * 3 + 0) * TPI + off, TS), :, :],
                    outsem[s]),
                pltpu.async_copy(
                    gbuf[s],
                    out_hbm.at[pl.ds((b * 3 + 1) * TPI + off, TS), :, :],
                    outsem[s]),
                pltpu.async_copy(
                    bbuf[s],
                    out_hbm.at[pl.ds((b * 3 + 2) * TPI + off, TS), :, :],
                    outsem[s]),
            ]
        for ds_ in out_descs:
            if ds_ is not None:
                for d in ds_:
                    d.wait()

    return k(x4, cmapf)


def kernel(x, cmap):
    x4 = x.reshape(B * TPI, 8, 128)
    cmapf = cmap.reshape(-1)
    out = _sc_colorize(x4, cmapf)
    return out.reshape(B, 3, H, W)


# speedup vs baseline: 264.1171x; 1.9547x over previous
import functools

import jax
import jax.numpy as jnp
from jax import lax
from jax.experimental import pallas as pl
from jax.experimental.pallas import tpu as pltpu
from jax.experimental.pallas import tpu_sc as plsc

B, H, W = 16, 512, 512
NW = 32
SR = 8
STRIPE = SR * W
N_STRIPES = (H // 2) // SR
L = 16
NCOLORS = 2048


def _sc_colorize(x, cmapf):
    mesh = plsc.VectorSubcoreMesh(core_axis_name="c", subcore_axis_name="s")

    @functools.partial(
        pl.kernel,
        out_type=jax.ShapeDtypeStruct((B, 3, H, W), jnp.float32),
        mesh=mesh,
        compiler_params=pltpu.CompilerParams(needs_layout_passes=False),
        scratch_types=[
            pltpu.VMEM((3 * NCOLORS,), jnp.float32),
            [pltpu.VMEM((1, SR, W), jnp.int32)] * 2,
            [pltpu.VMEM((1, 1, SR, W), jnp.float32)] * 2,
            [pltpu.VMEM((1, 1, SR, W), jnp.float32)] * 2,
            [pltpu.VMEM((1, 1, SR, W), jnp.float32)] * 2,
            [pltpu.SemaphoreType.DMA] * 2,
            [pltpu.SemaphoreType.DMA] * 2,
        ],
    )
    def k(x_hbm, cmap_hbm, out_hbm, tab, idxv, rbuf, gbuf, bbuf, insem,
          outsem):
        wid = lax.axis_index("s") * 2 + lax.axis_index("c")
        b = wid // 2
        row0 = (wid % 2) * (H // 2)

        pltpu.sync_copy(cmap_hbm, tab)

        in_desc = [None, None]
        out_descs = [None, None]

        def start_in(c):
            s = c & 1
            in_desc[s] = pltpu.async_copy(
                x_hbm.at[pl.ds(b, 1), pl.ds(row0 + c * SR, SR), :],
                idxv[s], insem[s])

        start_in(0)
        for c in range(N_STRIPES):
            s = c & 1
            if c + 1 < N_STRIPES:
                start_in(c + 1)
            in_desc[s].wait()
            if out_descs[s] is not None:
                for d in out_descs[s]:
                    d.wait()

            @plsc.parallel_loop(0, STRIPE, step=L, unroll=8)
            def body(i):
                r = i // W
                cc = i % W
                i3 = idxv[s][0, r, pl.ds(cc, L)] * 3
                rbuf[s][0, 0, r, pl.ds(cc, L)] = plsc.load_gather(tab, [i3])
                gbuf[s][0, 0, r, pl.ds(cc, L)] = plsc.load_gather(
                    tab, [i3 + 1])
                bbuf[s][0, 0, r, pl.ds(cc, L)] = plsc.load_gather(
                    tab, [i3 + 2])

            rsl = pl.ds(row0 + c * SR, SR)
            out_descs[s] = [
                pltpu.async_copy(
                    rbuf[s],
                    out_hbm.at[pl.ds(b, 1), pl.ds(0, 1), rsl, :], outsem[s]),
                pltpu.async_copy(
                    gbuf[s],
                    out_hbm.at[pl.ds(b, 1), pl.ds(1, 1), rsl, :], outsem[s]),
                pltpu.async_copy(
                    bbuf[s],
                    out_hbm.at[pl.ds(b, 1), pl.ds(2, 1), rsl, :], outsem[s]),
            ]
        for ds_ in out_descs:
            if ds_ is not None:
                for d in ds_:
                    d.wait()

    return k(x, cmapf)


def kernel(x, cmap):
    return _sc_colorize(x, cmap.reshape(-1))


# speedup vs baseline: 297.5018x; 1.1264x over previous
import functools

import jax
import jax.numpy as jnp
from jax import lax
from jax.experimental import pallas as pl
from jax.experimental.pallas import tpu as pltpu
from jax.experimental.pallas import tpu_sc as plsc

B, H, W = 16, 512, 512
NW = 32
SR = 16
STRIPE = SR * W
N_STRIPES = (H // 2) // SR
L = 16
NCOLORS = 2048


def _sc_colorize(x, cmapf):
    mesh = plsc.VectorSubcoreMesh(core_axis_name="c", subcore_axis_name="s")

    @functools.partial(
        pl.kernel,
        out_type=jax.ShapeDtypeStruct((B, 3, H, W), jnp.float32),
        mesh=mesh,
        compiler_params=pltpu.CompilerParams(needs_layout_passes=False),
        scratch_types=[
            pltpu.VMEM((3 * NCOLORS,), jnp.float32),
            [pltpu.VMEM((1, SR, W), jnp.int32)] * 2,
            [pltpu.VMEM((1, 1, SR, W), jnp.float32)] * 2,
            [pltpu.VMEM((1, 1, SR, W), jnp.float32)] * 2,
            [pltpu.VMEM((1, 1, SR, W), jnp.float32)] * 2,
            [pltpu.SemaphoreType.DMA] * 2,
            [pltpu.SemaphoreType.DMA] * 2,
        ],
    )
    def k(x_hbm, cmap_hbm, out_hbm, tab, idxv, rbuf, gbuf, bbuf, insem,
          outsem):
        wid = lax.axis_index("s") * 2 + lax.axis_index("c")
        b = wid // 2
        row0 = (wid % 2) * (H // 2)

        pltpu.sync_copy(cmap_hbm, tab)

        in_desc = [None, None]
        out_descs = [None, None]

        def start_in(c):
            s = c & 1
            in_desc[s] = pltpu.async_copy(
                x_hbm.at[pl.ds(b, 1), pl.ds(row0 + c * SR, SR), :],
                idxv[s], insem[s])

        start_in(0)
        for c in range(N_STRIPES):
            s = c & 1
            if c + 1 < N_STRIPES:
                start_in(c + 1)
            in_desc[s].wait()
            if out_descs[s] is not None:
                for d in out_descs[s]:
                    d.wait()

            @plsc.parallel_loop(0, STRIPE, step=L, unroll=16)
            def body(i):
                r = i // W
                cc = i % W
                i3 = idxv[s][0, r, pl.ds(cc, L)] * 3
                rbuf[s][0, 0, r, pl.ds(cc, L)] = plsc.load_gather(tab, [i3])
                gbuf[s][0, 0, r, pl.ds(cc, L)] = plsc.load_gather(
                    tab, [i3 + 1])
                bbuf[s][0, 0, r, pl.ds(cc, L)] = plsc.load_gather(
                    tab, [i3 + 2])

            rsl = pl.ds(row0 + c * SR, SR)
            out_descs[s] = [
                pltpu.async_copy(
                    rbuf[s],
                    out_hbm.at[pl.ds(b, 1), pl.ds(0, 1), rsl, :], outsem[s]),
                pltpu.async_copy(
                    gbuf[s],
                    out_hbm.at[pl.ds(b, 1), pl.ds(1, 1), rsl, :], outsem[s]),
                pltpu.async_copy(
                    bbuf[s],
                    out_hbm.at[pl.ds(b, 1), pl.ds(2, 1), rsl, :], outsem[s]),
            ]
        for ds_ in out_descs:
            if ds_ is not None:
                for d in ds_:
                    d.wait()

    return k(x, cmapf)


def kernel(x, cmap):
    return _sc_colorize(x, cmap.reshape(-1))
